# baseline (speedup 1.0000x reference)
import jax
import jax.numpy as jnp
from jax.experimental import pallas as pl
from jax.experimental.pallas import tpu as pltpu

_NMS_THRESH = 0.5
_LANES = 128
_SUBLANES = 8
_BIG = 1.0e9


_KEPT = 2.0e9
_SUPP = 3.0e9


def _nms_kernel(n_boxes, x1_ref, y1_ref, x2_ref, y2_ref, keep_ref,
                area_ref, m_ref, pos_ref):
    shape = x1_ref.shape
    rows = shape[0]
    f32 = jnp.float32

    pos0 = (jax.lax.broadcasted_iota(jnp.int32, shape, 0) * _LANES
            + jax.lax.broadcasted_iota(jnp.int32, shape, 1)).astype(f32)
    pos_ref[...] = pos0
    valid = pos0 < f32(n_boxes)
    area_ref[...] = (x2_ref[...] - x1_ref[...]) * (y2_ref[...] - y1_ref[...])
    m_ref[...] = jnp.where(valid, pos0, f32(_SUPP))

    lane_iota = jax.lax.broadcasted_iota(
        jnp.int32, (1, _LANES), 1).astype(f32)

    def pick(p):
        rf = jnp.floor(p * (1.0 / _LANES))
        r = jnp.minimum(rf.astype(jnp.int32), rows - 1)
        cf = p - rf * _LANES
        onehot = jnp.where(lane_iota == cf, f32(1.0), f32(0.0))
        return (jnp.sum(x1_ref[pl.ds(r, 1), :] * onehot),
                jnp.sum(y1_ref[pl.ds(r, 1), :] * onehot),
                jnp.sum(x2_ref[pl.ds(r, 1), :] * onehot),
                jnp.sum(y2_ref[pl.ds(r, 1), :] * onehot),
                jnp.sum(area_ref[pl.ds(r, 1), :] * onehot))

    def iou_terms(x1i, y1i, x2i, y2i, ai, x1, y1, x2, y2, area):
        xx1 = jnp.maximum(x1i, x1)
        yy1 = jnp.maximum(y1i, y1)
        xx2 = jnp.minimum(x2i, x2)
        yy2 = jnp.minimum(y2i, y2)
        w = jnp.maximum(f32(1e-10), xx2 - xx1)
        h = jnp.maximum(f32(1e-10), yy2 - yy1)
        inter = w * h
        return inter / (ai + area - inter + f32(1e-14))

    def comb(a, b):
        a1, a2 = a
        b1, b2 = b
        return (jnp.minimum(a1, b1),
                jnp.minimum(jnp.maximum(a1, b1), jnp.minimum(a2, b2)))

    def two_smallest(m):
        nc = m.shape[0] // 8
        leaves = [m[i * 8:(i + 1) * 8] for i in range(nc)]
        ts = [(jnp.minimum(leaves[i], leaves[i + 1]),
               jnp.maximum(leaves[i], leaves[i + 1]))
              for i in range(0, nc - 1, 2)]
        if nc % 2:
            ts.append((leaves[-1], jnp.full((8, _LANES), _SUPP, f32)))
        while len(ts) > 1:
            nxt = [comb(ts[i], ts[i + 1]) for i in range(0, len(ts) - 1, 2)]
            if len(ts) % 2:
                nxt.append(ts[-1])
            ts = nxt
        t0, t1 = ts[0]
        s1 = jnp.min(t0)
        s2 = jnp.min(jnp.where(t0 == s1, t1, t0))
        return s1, s2

    ones_row = jnp.ones((1, _LANES), f32)

    def cond(carry):
        return carry[0] < _BIG * 0.5

    def body(carry):
        p1, p2 = carry
        b1 = pick(p1)
        b2 = pick(p2)

        iou12 = iou_terms(*b1, ones_row * b2[0], ones_row * b2[1],
                          ones_row * b2[2], ones_row * b2[3],
                          ones_row * b2[4])
        p2_acts = jnp.logical_and(p2 < _BIG * 0.5,
                                  jnp.logical_not(iou12 > _NMS_THRESH))

        x1 = x1_ref[...]
        y1 = y1_ref[...]
        x2 = x2_ref[...]
        y2 = y2_ref[...]
        area = area_ref[...]
        pos = pos_ref[...]

        iou1 = iou_terms(*b1, x1, y1, x2, y2, area)
        iou2 = iou_terms(*b2, x1, y1, x2, y2, area)
        sup = (iou1 > _NMS_THRESH) & (pos > p1)
        sup2 = (iou2 > _NMS_THRESH) & (pos > p2) & p2_acts
        sup = sup | sup2
        picked = (pos == p1) | ((pos == p2) & p2_acts)
        m = jnp.where(sup, f32(_SUPP),
                      jnp.where(picked, f32(_KEPT), m_ref[...]))
        m_ref[...] = m
        return two_smallest(m)

    jax.lax.while_loop(cond, body, two_smallest(m_ref[...]))
    keep_ref[...] = jnp.where(m_ref[...] == f32(_KEPT), f32(1.0), f32(0.0))


def kernel(boxes, scores):
    n = boxes.shape[0]
    pad_n = ((n + _LANES * _SUBLANES - 1)
             // (_LANES * _SUBLANES)) * (_LANES * _SUBLANES)
    rows = pad_n // _LANES

    xy1 = boxes[:, :2] - boxes[:, 2:] * 0.5
    xy2 = boxes[:, :2] + boxes[:, 2:] * 0.5
    boxes_xyxy = jnp.concatenate([xy1, xy2], axis=-1)

    order = jnp.argsort(-scores)
    b = boxes_xyxy[order]
    planes = [
        jnp.pad(b[:, k], (0, pad_n - n)).reshape(rows, _LANES)
        for k in range(4)
    ]

    keep_sorted = pl.pallas_call(
        lambda *refs: _nms_kernel(n, *refs),
        out_shape=jax.ShapeDtypeStruct((rows, _LANES), jnp.float32),
        scratch_shapes=[
            pltpu.VMEM((rows, _LANES), jnp.float32),
            pltpu.VMEM((rows, _LANES), jnp.float32),
            pltpu.VMEM((rows, _LANES), jnp.float32),
        ],
    )(*planes)

    keep_s = keep_sorted.reshape(-1)[:n]
    keep = jnp.zeros((n,), boxes.dtype).at[order].set(keep_s)
    out = jnp.concatenate(
        [boxes_xyxy * keep[:, None], (scores * keep)[:, None]], axis=-1)
    return out


# speedup vs baseline: 1.0333x; 1.0333x over previous
import jax
import jax.numpy as jnp
from jax.experimental import pallas as pl
from jax.experimental.pallas import tpu as pltpu

_NMS_THRESH = 0.5
_LANES = 128
_SUBLANES = 8
_BIG = 1.0e9


_KEPT = 2.0e9
_SUPP = 3.0e9


def _nms_kernel(n_boxes, x1_ref, y1_ref, x2_ref, y2_ref, keep_ref,
                area_ref, m_ref, pos_ref):
    shape = x1_ref.shape
    rows = shape[0]
    f32 = jnp.float32

    pos0 = (jax.lax.broadcasted_iota(jnp.int32, shape, 0) * _LANES
            + jax.lax.broadcasted_iota(jnp.int32, shape, 1)).astype(f32)
    pos_ref[...] = pos0
    valid = pos0 < f32(n_boxes)
    area_ref[...] = (x2_ref[...] - x1_ref[...]) * (y2_ref[...] - y1_ref[...])
    m_ref[...] = jnp.where(valid, pos0, f32(_SUPP))

    lane_iota = jax.lax.broadcasted_iota(
        jnp.int32, (1, _LANES), 1).astype(f32)

    def pick(p):
        rf = jnp.floor(p * (1.0 / _LANES))
        r = jnp.minimum(rf.astype(jnp.int32), rows - 1)
        cf = p - rf * _LANES
        onehot = jnp.where(lane_iota == cf, f32(1.0), f32(0.0))
        return (jnp.sum(x1_ref[pl.ds(r, 1), :] * onehot),
                jnp.sum(y1_ref[pl.ds(r, 1), :] * onehot),
                jnp.sum(x2_ref[pl.ds(r, 1), :] * onehot),
                jnp.sum(y2_ref[pl.ds(r, 1), :] * onehot),
                jnp.sum(area_ref[pl.ds(r, 1), :] * onehot))

    def iou_terms(x1i, y1i, x2i, y2i, ai, x1, y1, x2, y2, area):
        xx1 = jnp.maximum(x1i, x1)
        yy1 = jnp.maximum(y1i, y1)
        xx2 = jnp.minimum(x2i, x2)
        yy2 = jnp.minimum(y2i, y2)
        w = jnp.maximum(f32(1e-10), xx2 - xx1)
        h = jnp.maximum(f32(1e-10), yy2 - yy1)
        inter = w * h
        return inter / (ai + area - inter + f32(1e-14))

    ones_row = jnp.ones((1, _LANES), f32)

    def cond(p1):
        return p1 < _BIG * 0.5

    def body(p1):
        m_cur = m_ref[...]
        p2 = jnp.min(jnp.where(m_cur == p1, f32(_SUPP), m_cur))
        b1 = pick(p1)
        b2 = pick(p2)

        iou12 = iou_terms(*b1, ones_row * b2[0], ones_row * b2[1],
                          ones_row * b2[2], ones_row * b2[3],
                          ones_row * b2[4])
        p2_acts = jnp.logical_and(p2 < _BIG * 0.5,
                                  jnp.logical_not(iou12 > _NMS_THRESH))

        x1 = x1_ref[...]
        y1 = y1_ref[...]
        x2 = x2_ref[...]
        y2 = y2_ref[...]
        area = area_ref[...]
        pos = pos_ref[...]

        iou1 = iou_terms(*b1, x1, y1, x2, y2, area)
        iou2 = iou_terms(*b2, x1, y1, x2, y2, area)
        sup = (iou1 > _NMS_THRESH) & (pos > p1)
        sup2 = (iou2 > _NMS_THRESH) & (pos > p2) & p2_acts
        sup = sup | sup2
        picked = (pos == p1) | ((pos == p2) & p2_acts)
        m = jnp.where(sup, f32(_SUPP),
                      jnp.where(picked, f32(_KEPT), m_cur))
        m_ref[...] = m
        return jnp.min(m)

    jax.lax.while_loop(cond, body, jnp.min(m_ref[...]))
    keep_ref[...] = jnp.where(m_ref[...] == f32(_KEPT), f32(1.0), f32(0.0))


def kernel(boxes, scores):
    n = boxes.shape[0]
    pad_n = ((n + _LANES * _SUBLANES - 1)
             // (_LANES * _SUBLANES)) * (_LANES * _SUBLANES)
    rows = pad_n // _LANES

    xy1 = boxes[:, :2] - boxes[:, 2:] * 0.5
    xy2 = boxes[:, :2] + boxes[:, 2:] * 0.5
    boxes_xyxy = jnp.concatenate([xy1, xy2], axis=-1)

    order = jnp.argsort(-scores)
    b = boxes_xyxy[order]
    planes = [
        jnp.pad(b[:, k], (0, pad_n - n)).reshape(rows, _LANES)
        for k in range(4)
    ]

    keep_sorted = pl.pallas_call(
        lambda *refs: _nms_kernel(n, *refs),
        out_shape=jax.ShapeDtypeStruct((rows, _LANES), jnp.float32),
        scratch_shapes=[
            pltpu.VMEM((rows, _LANES), jnp.float32),
            pltpu.VMEM((rows, _LANES), jnp.float32),
            pltpu.VMEM((rows, _LANES), jnp.float32),
        ],
    )(*planes)

    keep_s = keep_sorted.reshape(-1)[:n]
    keep = jnp.zeros((n,), boxes.dtype).at[order].set(keep_s)
    out = jnp.concatenate(
        [boxes_xyxy * keep[:, None], (scores * keep)[:, None]], axis=-1)
    return out


# speedup vs baseline: 1.0440x; 1.0103x over previous
import jax
import jax.numpy as jnp
from jax.experimental import pallas as pl
from jax.experimental.pallas import tpu as pltpu

_NMS_THRESH = 0.5
_LANES = 128
_SUBLANES = 8
_BIG = 1.0e9


_KEPT = 2.0e9
_SUPP = 3.0e9


def _nms_kernel(n_boxes, x1_ref, y1_ref, x2_ref, y2_ref, keep_ref,
                area_ref, m_ref, pos_ref):
    shape = x1_ref.shape
    rows = shape[0]
    f32 = jnp.float32

    pos0 = (jax.lax.broadcasted_iota(jnp.int32, shape, 0) * _LANES
            + jax.lax.broadcasted_iota(jnp.int32, shape, 1)).astype(f32)
    pos_ref[...] = pos0
    valid = pos0 < f32(n_boxes)
    area_ref[...] = (x2_ref[...] - x1_ref[...]) * (y2_ref[...] - y1_ref[...])
    m_ref[...] = jnp.where(valid, pos0, f32(_SUPP))

    lane_iota = jax.lax.broadcasted_iota(
        jnp.int32, (1, _LANES), 1).astype(f32)

    def pick(p):
        rf = jnp.floor(p * (1.0 / _LANES))
        r = jnp.minimum(rf.astype(jnp.int32), rows - 1)
        cf = p - rf * _LANES
        onehot = jnp.where(lane_iota == cf, f32(1.0), f32(0.0))
        return (jnp.sum(x1_ref[pl.ds(r, 1), :] * onehot),
                jnp.sum(y1_ref[pl.ds(r, 1), :] * onehot),
                jnp.sum(x2_ref[pl.ds(r, 1), :] * onehot),
                jnp.sum(y2_ref[pl.ds(r, 1), :] * onehot),
                jnp.sum(area_ref[pl.ds(r, 1), :] * onehot))

    def iou_terms(x1i, y1i, x2i, y2i, ai, x1, y1, x2, y2, area):
        xx1 = jnp.maximum(x1i, x1)
        yy1 = jnp.maximum(y1i, y1)
        xx2 = jnp.minimum(x2i, x2)
        yy2 = jnp.minimum(y2i, y2)
        w = jnp.maximum(f32(1e-10), xx2 - xx1)
        h = jnp.maximum(f32(1e-10), yy2 - yy1)
        inter = w * h
        return inter / (ai + area - inter + f32(1e-14))

    ones_row = jnp.ones((1, _LANES), f32)

    def cond(p1):
        return p1 < _BIG * 0.5

    def splat_iou(bi, bj):
        return iou_terms(*bi, ones_row * bj[0], ones_row * bj[1],
                         ones_row * bj[2], ones_row * bj[3],
                         ones_row * bj[4])

    def body(p1):
        m_cur = m_ref[...]
        m2 = jnp.where(m_cur == p1, f32(_SUPP), m_cur)
        p2 = jnp.min(m2)
        m3 = jnp.where(m2 == p2, f32(_SUPP), m2)
        p3 = jnp.min(m3)
        p4 = jnp.min(jnp.where(m3 == p3, f32(_SUPP), m3))
        b1 = pick(p1)
        b2 = pick(p2)
        b3 = pick(p3)
        b4 = pick(p4)

        nsup = lambda i, j: jnp.logical_not(splat_iou(i, j) > _NMS_THRESH)
        a2 = (p2 < _BIG * 0.5) & nsup(b1, b2)
        a3 = ((p3 < _BIG * 0.5) & nsup(b1, b3)
              & jnp.logical_not(a2 & (splat_iou(b2, b3) > _NMS_THRESH)))
        a4 = ((p4 < _BIG * 0.5) & nsup(b1, b4)
              & jnp.logical_not(a2 & (splat_iou(b2, b4) > _NMS_THRESH))
              & jnp.logical_not(a3 & (splat_iou(b3, b4) > _NMS_THRESH)))

        x1 = x1_ref[...]
        y1 = y1_ref[...]
        x2 = x2_ref[...]
        y2 = y2_ref[...]
        area = area_ref[...]
        pos = pos_ref[...]

        iou1 = iou_terms(*b1, x1, y1, x2, y2, area)
        iou2 = iou_terms(*b2, x1, y1, x2, y2, area)
        iou3 = iou_terms(*b3, x1, y1, x2, y2, area)
        iou4 = iou_terms(*b4, x1, y1, x2, y2, area)
        sup = ((iou1 > _NMS_THRESH) & (pos > p1)
               | (iou2 > _NMS_THRESH) & (pos > p2) & a2
               | (iou3 > _NMS_THRESH) & (pos > p3) & a3
               | (iou4 > _NMS_THRESH) & (pos > p4) & a4)
        picked = ((pos == p1) | ((pos == p2) & a2)
                  | ((pos == p3) & a3) | ((pos == p4) & a4))
        m = jnp.where(sup, f32(_SUPP),
                      jnp.where(picked, f32(_KEPT), m_cur))
        m_ref[...] = m
        return jnp.min(m)

    jax.lax.while_loop(cond, body, jnp.min(m_ref[...]))
    keep_ref[...] = jnp.where(m_ref[...] == f32(_KEPT), f32(1.0), f32(0.0))


def kernel(boxes, scores):
    n = boxes.shape[0]
    pad_n = ((n + _LANES * _SUBLANES - 1)
             // (_LANES * _SUBLANES)) * (_LANES * _SUBLANES)
    rows = pad_n // _LANES

    xy1 = boxes[:, :2] - boxes[:, 2:] * 0.5
    xy2 = boxes[:, :2] + boxes[:, 2:] * 0.5
    boxes_xyxy = jnp.concatenate([xy1, xy2], axis=-1)

    order = jnp.argsort(-scores)
    b = boxes_xyxy[order]
    planes = [
        jnp.pad(b[:, k], (0, pad_n - n)).reshape(rows, _LANES)
        for k in range(4)
    ]

    keep_sorted = pl.pallas_call(
        lambda *refs: _nms_kernel(n, *refs),
        out_shape=jax.ShapeDtypeStruct((rows, _LANES), jnp.float32),
        scratch_shapes=[
            pltpu.VMEM((rows, _LANES), jnp.float32),
            pltpu.VMEM((rows, _LANES), jnp.float32),
            pltpu.VMEM((rows, _LANES), jnp.float32),
        ],
    )(*planes)

    keep_s = keep_sorted.reshape(-1)[:n]
    keep = jnp.zeros((n,), boxes.dtype).at[order].set(keep_s)
    out = jnp.concatenate(
        [boxes_xyxy * keep[:, None], (scores * keep)[:, None]], axis=-1)
    return out
